# baseline (speedup 1.0000x reference)
import functools

import jax
import jax.numpy as jnp
from jax import lax
from jax.experimental import pallas as pl
from jax.experimental.pallas import tpu as pltpu
from jax.experimental.pallas import tpu_sc as plsc

N_NODES = 10000
N_EDGES = 320000
IN_CH = 128
HID = 128
OUT_CH = 16
KAN_MID = 64
N_COEFF = 8

NC = 2
NS = 16
CHUNK = 128
EDGES_PER_TILE = 10240
N_CHUNKS = EDGES_PER_TILE // CHUNK
E_PAD = 327680
CHUNK_CNT = 128
N_CHUNKS_CNT = E_PAD // (NC * NS * CHUNK_CNT)
N_PAD = 10240
ROWS_PER_TILE = N_PAD // NS


def _sc_agg_body(x_hbm, src_hbm, dst_hbm, zrows_hbm,
                 sums_hbm,
                 src_v0, dst_v0, src_v1, dst_v1, rows0, rows1, acc_s,
                 gsem0, gsem1, isem0, isem1):
    c = lax.axis_index("c")
    s = lax.axis_index("s")
    row0 = s * ROWS_PER_TILE
    pltpu.sync_copy(zrows_hbm, acc_s.at[pl.ds(row0, ROWS_PER_TILE)])
    plsc.subcore_barrier()
    ebase = (c * NS + s) * EDGES_PER_TILE

    def idx_drain(sem):
        pltpu.make_async_copy(src_hbm.at[pl.ds(0, CHUNK)], src_v0, sem).wait()
        pltpu.make_async_copy(src_hbm.at[pl.ds(0, CHUNK)], src_v0, sem).wait()

    def g_drain(sem, rows):
        pltpu.make_async_copy(x_hbm.at[pl.ds(0, CHUNK)], rows, sem).wait()

    pltpu.async_copy(src_hbm.at[pl.ds(ebase, CHUNK)], src_v0, isem0)
    pltpu.async_copy(dst_hbm.at[pl.ds(ebase, CHUNK)], dst_v0, isem0)
    pltpu.async_copy(src_hbm.at[pl.ds(ebase + CHUNK, CHUNK)], src_v1, isem1)
    pltpu.async_copy(dst_hbm.at[pl.ds(ebase + CHUNK, CHUNK)], dst_v1, isem1)
    idx_drain(isem0)
    pltpu.async_copy(x_hbm.at[src_v0], rows0, gsem0)

    def pair(p, carry):
        j0 = 2 * p
        g_drain(gsem0, rows0)
        idx_drain(isem1)
        pltpu.async_copy(x_hbm.at[src_v1], rows1, gsem1)
        pltpu.sync_copy(rows0, acc_s.at[dst_v0], add=True)
        off0 = ebase + (j0 + 2) * CHUNK
        pltpu.async_copy(src_hbm.at[pl.ds(off0, CHUNK)], src_v0, isem0)
        pltpu.async_copy(dst_hbm.at[pl.ds(off0, CHUNK)], dst_v0, isem0)
        g_drain(gsem1, rows1)
        idx_drain(isem0)
        pltpu.async_copy(x_hbm.at[src_v0], rows0, gsem0)
        pltpu.sync_copy(rows1, acc_s.at[dst_v1], add=True)
        off1 = ebase + (j0 + 3) * CHUNK
        pltpu.async_copy(src_hbm.at[pl.ds(off1, CHUNK)], src_v1, isem1)
        pltpu.async_copy(dst_hbm.at[pl.ds(off1, CHUNK)], dst_v1, isem1)
        return carry

    lax.fori_loop(0, N_CHUNKS // 2 - 1, pair, 0)
    g_drain(gsem0, rows0)
    idx_drain(isem1)
    pltpu.async_copy(x_hbm.at[src_v1], rows1, gsem1)
    pltpu.sync_copy(rows0, acc_s.at[dst_v0], add=True)
    g_drain(gsem1, rows1)
    pltpu.sync_copy(rows1, acc_s.at[dst_v1], add=True)
    plsc.subcore_barrier()
    pltpu.sync_copy(acc_s.at[pl.ds(row0, ROWS_PER_TILE)],
                    sums_hbm.at[c, pl.ds(row0, ROWS_PER_TILE)])


def _sc_cnt_body(dst3_hbm, zrows_hbm, ones_hbm,
                 cnt_hbm,
                 dst_all, ones_v, cnt_s, ssem):
    c = lax.axis_index("c")
    s = lax.axis_index("s")
    w = c * NS + s
    row0 = s * ROWS_PER_TILE
    pltpu.sync_copy(zrows_hbm, cnt_s.at[pl.ds(row0, ROWS_PER_TILE)])
    pltpu.sync_copy(ones_hbm, ones_v)
    pltpu.sync_copy(dst3_hbm.at[w], dst_all)
    plsc.subcore_barrier()

    def batch(b, carry):
        j0 = b * 8
        for k in range(8):
            pltpu.async_copy(ones_v, cnt_s.at[dst_all.at[j0 + k]], ssem,
                             add=True)
        for k in range(8):
            pltpu.make_async_copy(ones_hbm, ones_v, ssem).wait()
        return carry

    lax.fori_loop(0, N_CHUNKS_CNT // 8, batch, 0)
    plsc.subcore_barrier()
    pltpu.sync_copy(cnt_s.at[pl.ds(row0, ROWS_PER_TILE)],
                    cnt_hbm.at[c, pl.ds(row0, ROWS_PER_TILE)])


@functools.lru_cache(maxsize=None)
def _sc_kernels():
    mesh = plsc.VectorSubcoreMesh(core_axis_name="c", subcore_axis_name="s",
                                  num_cores=NC, num_subcores=NS)
    agg = pl.kernel(
        _sc_agg_body,
        out_type=jax.ShapeDtypeStruct((NC, N_PAD, HID), jnp.float32),
        mesh=mesh,
        scratch_types=[
            pltpu.VMEM((CHUNK,), jnp.int32),
            pltpu.VMEM((CHUNK,), jnp.int32),
            pltpu.VMEM((CHUNK,), jnp.int32),
            pltpu.VMEM((CHUNK,), jnp.int32),
            pltpu.VMEM((CHUNK, HID), jnp.float32),
            pltpu.VMEM((CHUNK, HID), jnp.float32),
            pltpu.VMEM_SHARED((N_PAD, HID), jnp.float32),
            pltpu.SemaphoreType.DMA,
            pltpu.SemaphoreType.DMA,
            pltpu.SemaphoreType.DMA,
            pltpu.SemaphoreType.DMA,
        ],
    )
    cntk = pl.kernel(
        _sc_cnt_body,
        out_type=jax.ShapeDtypeStruct((NC, N_PAD, HID), jnp.float32),
        mesh=mesh,
        scratch_types=[
            pltpu.VMEM((N_CHUNKS_CNT, CHUNK_CNT), jnp.int32),
            pltpu.VMEM((CHUNK_CNT, HID), jnp.float32),
            pltpu.VMEM_SHARED((N_PAD, HID), jnp.float32),
            pltpu.SemaphoreType.DMA,
        ],
    )
    return agg, cntk


ROW_BLK = 1024
N_BLKS = N_PAD // ROW_BLK


def _combine_body(sums_ref, cnt_ref, x_ref, wlT_ref, b_ref, wrT_ref, o_ref):
    sm = sums_ref[0] + sums_ref[1]
    c = cnt_ref[0, :, :1] + cnt_ref[1, :, :1]
    mean = sm / jnp.maximum(c, 1.0)
    h = (jnp.dot(mean, wlT_ref[...], preferred_element_type=jnp.float32)
         + b_ref[...]
         + jnp.dot(x_ref[...], wrT_ref[...], preferred_element_type=jnp.float32))
    o_ref[...] = jnp.maximum(h, 0.0)


def _combine(sums, cnt, x, wlT, b_row, wrT):
    return pl.pallas_call(
        _combine_body,
        grid=(N_BLKS,),
        in_specs=[
            pl.BlockSpec((NC, ROW_BLK, HID), lambda i: (0, i, 0)),
            pl.BlockSpec((NC, ROW_BLK, HID), lambda i: (0, i, 0)),
            pl.BlockSpec((ROW_BLK, IN_CH), lambda i: (i, 0)),
            pl.BlockSpec((IN_CH, HID), lambda i: (0, 0)),
            pl.BlockSpec((1, HID), lambda i: (0, 0)),
            pl.BlockSpec((IN_CH, HID), lambda i: (0, 0)),
        ],
        out_specs=pl.BlockSpec((ROW_BLK, HID), lambda i: (i, 0)),
        out_shape=jax.ShapeDtypeStruct((N_PAD, HID), jnp.float32),
    )(sums, cnt, x, wlT, b_row, wrT)


def _spline_mix(h, swT_ref, ssT_ref, out_width):
    u = h * 2.5 + 5.5
    mf = jnp.floor(u)
    t = u - mf
    t2 = t * t
    t3 = t2 * t
    w = 1.0 - t
    seg0 = t3 * (1.0 / 6.0)
    seg1 = (-3.0 * t3 + 3.0 * t2 + 3.0 * t + 1.0) * (1.0 / 6.0)
    seg2 = (3.0 * t3 - 6.0 * t2 + 4.0) * (1.0 / 6.0)
    seg3 = (w * w * w) * (1.0 / 6.0)
    ss = ssT_ref[...]
    acc = jnp.zeros((h.shape[0], out_width), jnp.float32)
    for j in range(N_COEFF):
        d = mf - float(j)
        bj = jnp.where(
            d == 0.0, seg0,
            jnp.where(d == 1.0, seg1,
                      jnp.where(d == 2.0, seg2,
                                jnp.where(d == 3.0, seg3, 0.0))))
        acc = acc + jnp.dot(bj, swT_ref[j] * ss,
                            preferred_element_type=jnp.float32)
    return acc


def _silu(v):
    return v / (1.0 + jnp.exp(-v))


def _kan_body(h_ref, bw1T_ref, sw1T_ref, ss1T_ref, bw2T_ref, sw2T_ref,
              ss2T_ref, o_ref):
    h = h_ref[...]
    mid = jnp.dot(_silu(h), bw1T_ref[...], preferred_element_type=jnp.float32)
    mid = mid + _spline_mix(h, sw1T_ref, ss1T_ref, KAN_MID)
    out = jnp.dot(_silu(mid), bw2T_ref[...], preferred_element_type=jnp.float32)
    out = out + _spline_mix(mid, sw2T_ref, ss2T_ref, OUT_CH)
    o_ref[...] = out


def _kan(h, bw1T, sw1T, ss1T, bw2T, sw2T, ss2T):
    return pl.pallas_call(
        _kan_body,
        grid=(N_BLKS,),
        in_specs=[
            pl.BlockSpec((ROW_BLK, HID), lambda i: (i, 0)),
            pl.BlockSpec((HID, KAN_MID), lambda i: (0, 0)),
            pl.BlockSpec((N_COEFF, HID, KAN_MID), lambda i: (0, 0, 0)),
            pl.BlockSpec((HID, KAN_MID), lambda i: (0, 0)),
            pl.BlockSpec((KAN_MID, OUT_CH), lambda i: (0, 0)),
            pl.BlockSpec((N_COEFF, KAN_MID, OUT_CH), lambda i: (0, 0, 0)),
            pl.BlockSpec((KAN_MID, OUT_CH), lambda i: (0, 0)),
        ],
        out_specs=pl.BlockSpec((ROW_BLK, OUT_CH), lambda i: (i, 0)),
        out_shape=jax.ShapeDtypeStruct((N_PAD, OUT_CH), jnp.float32),
    )(h, bw1T, sw1T, ss1T, bw2T, sw2T, ss2T)


def kernel(x, edge_index, W1l, b1, W1r, W2l, b2, W2r, bw1, sw1, ss1,
           bw2, sw2, ss2):
    ei = edge_index.astype(jnp.int32)
    xp = jnp.concatenate([x, jnp.zeros((N_PAD - N_NODES, IN_CH), jnp.float32)])
    zrows = jnp.zeros((ROWS_PER_TILE, HID), jnp.float32)
    ones128 = jnp.ones((CHUNK_CNT, HID), jnp.float32)

    sc_agg, sc_cnt = _sc_kernels()
    pad = E_PAD - N_EDGES
    dst3 = jnp.concatenate([ei[1], jnp.full((pad,), N_NODES, jnp.int32)]
                           ).reshape(NC * NS, N_CHUNKS_CNT, CHUNK_CNT)
    padv = jnp.full((pad,), N_NODES, jnp.int32)
    srcf = jnp.concatenate([ei[0], padv])
    dstf = jnp.concatenate([ei[1], padv])
    cnt = sc_cnt(dst3, zrows, ones128)
    sums1 = sc_agg(xp, srcf, dstf, zrows)
    h1 = _combine(sums1, cnt, xp, W1l.T, b1.reshape(1, HID), W1r.T)
    sums2 = sc_agg(h1, srcf, dstf, zrows)
    h2 = _combine(sums2, cnt, h1, W2l.T, b2.reshape(1, HID), W2r.T)
    out = _kan(h2, bw1.T, jnp.transpose(sw1, (2, 1, 0)), ss1.T,
               bw2.T, jnp.transpose(sw2, (2, 1, 0)), ss2.T)
    return out[:N_NODES]


# speedup vs baseline: 2.2911x; 2.2911x over previous
import functools

import jax
import jax.numpy as jnp
from jax import lax
from jax.experimental import pallas as pl
from jax.experimental.pallas import tpu as pltpu
from jax.experimental.pallas import tpu_sc as plsc

N_NODES = 10000
N_EDGES = 320000
IN_CH = 128
HID = 128
OUT_CH = 16
KAN_MID = 64
N_COEFF = 8

NC = 2
NS = 16
CHUNK = 80
EDGES_PER_TILE = N_EDGES // (NC * NS)
N_CHUNKS = EDGES_PER_TILE // CHUNK
E_PAD = 327680
CHUNK_CNT = 128
N_CHUNKS_CNT = E_PAD // (NC * NS * CHUNK_CNT)
N_PAD = 10240
ROWS_PER_TILE = N_PAD // NS


def _sc_agg_body(x_hbm, src_hbm, dst_hbm, zrows_hbm,
                 sums_hbm,
                 src_v0, dst_v0, src_v1, dst_v1, rows0, rows1, acc_s,
                 gsem0, gsem1, isem0, isem1):
    c = lax.axis_index("c")
    s = lax.axis_index("s")
    row0 = s * ROWS_PER_TILE
    pltpu.sync_copy(zrows_hbm, acc_s.at[pl.ds(row0, ROWS_PER_TILE)])
    plsc.subcore_barrier()
    ebase = (c * NS + s) * EDGES_PER_TILE

    def idx_drain(sem):
        pltpu.make_async_copy(src_hbm.at[pl.ds(0, CHUNK)], src_v0, sem).wait()
        pltpu.make_async_copy(src_hbm.at[pl.ds(0, CHUNK)], src_v0, sem).wait()

    def g_drain(sem, rows):
        pltpu.make_async_copy(x_hbm.at[pl.ds(0, CHUNK)], rows, sem).wait()

    pltpu.async_copy(src_hbm.at[pl.ds(ebase, CHUNK)], src_v0, isem0)
    pltpu.async_copy(dst_hbm.at[pl.ds(ebase, CHUNK)], dst_v0, isem0)
    pltpu.async_copy(src_hbm.at[pl.ds(ebase + CHUNK, CHUNK)], src_v1, isem1)
    pltpu.async_copy(dst_hbm.at[pl.ds(ebase + CHUNK, CHUNK)], dst_v1, isem1)
    idx_drain(isem0)
    pltpu.async_copy(x_hbm.at[src_v0], rows0, gsem0)

    def pair(p, carry):
        j0 = 2 * p
        g_drain(gsem0, rows0)
        idx_drain(isem1)
        pltpu.async_copy(x_hbm.at[src_v1], rows1, gsem1)
        pltpu.sync_copy(rows0, acc_s.at[dst_v0], add=True)
        off0 = ebase + (j0 + 2) * CHUNK
        pltpu.async_copy(src_hbm.at[pl.ds(off0, CHUNK)], src_v0, isem0)
        pltpu.async_copy(dst_hbm.at[pl.ds(off0, CHUNK)], dst_v0, isem0)
        g_drain(gsem1, rows1)
        idx_drain(isem0)
        pltpu.async_copy(x_hbm.at[src_v0], rows0, gsem0)
        pltpu.sync_copy(rows1, acc_s.at[dst_v1], add=True)
        off1 = ebase + (j0 + 3) * CHUNK
        pltpu.async_copy(src_hbm.at[pl.ds(off1, CHUNK)], src_v1, isem1)
        pltpu.async_copy(dst_hbm.at[pl.ds(off1, CHUNK)], dst_v1, isem1)
        return carry

    lax.fori_loop(0, (N_CHUNKS - 1) // 2, pair, 0)
    g_drain(gsem0, rows0)
    pltpu.sync_copy(rows0, acc_s.at[dst_v0], add=True)
    idx_drain(isem1)
    plsc.subcore_barrier()
    pltpu.sync_copy(acc_s.at[pl.ds(row0, ROWS_PER_TILE)],
                    sums_hbm.at[c, pl.ds(row0, ROWS_PER_TILE)])


def _sc_cnt_body(dst3_hbm, zrows_hbm, ones_hbm,
                 cnt_hbm,
                 dst_all, ones_v, cnt_s, ssem):
    c = lax.axis_index("c")
    s = lax.axis_index("s")
    w = c * NS + s
    row0 = s * ROWS_PER_TILE
    pltpu.sync_copy(zrows_hbm, cnt_s.at[pl.ds(row0, ROWS_PER_TILE)])
    pltpu.sync_copy(ones_hbm, ones_v)
    pltpu.sync_copy(dst3_hbm.at[w], dst_all)
    plsc.subcore_barrier()

    def batch(b, carry):
        j0 = b * 8
        for k in range(8):
            pltpu.async_copy(ones_v, cnt_s.at[dst_all.at[j0 + k]], ssem,
                             add=True)
        for k in range(8):
            pltpu.make_async_copy(ones_hbm, ones_v, ssem).wait()
        return carry

    lax.fori_loop(0, N_CHUNKS_CNT // 8, batch, 0)
    plsc.subcore_barrier()
    pltpu.sync_copy(cnt_s.at[pl.ds(row0, ROWS_PER_TILE)],
                    cnt_hbm.at[c, pl.ds(row0, ROWS_PER_TILE)])


@functools.lru_cache(maxsize=None)
def _sc_kernels():
    mesh = plsc.VectorSubcoreMesh(core_axis_name="c", subcore_axis_name="s",
                                  num_cores=NC, num_subcores=NS)
    agg = pl.kernel(
        _sc_agg_body,
        out_type=jax.ShapeDtypeStruct((NC, N_PAD, HID), jnp.float32),
        mesh=mesh,
        scratch_types=[
            pltpu.VMEM((CHUNK,), jnp.int32),
            pltpu.VMEM((CHUNK,), jnp.int32),
            pltpu.VMEM((CHUNK,), jnp.int32),
            pltpu.VMEM((CHUNK,), jnp.int32),
            pltpu.VMEM((CHUNK, HID), jnp.float32),
            pltpu.VMEM((CHUNK, HID), jnp.float32),
            pltpu.VMEM_SHARED((N_PAD, HID), jnp.float32),
            pltpu.SemaphoreType.DMA,
            pltpu.SemaphoreType.DMA,
            pltpu.SemaphoreType.DMA,
            pltpu.SemaphoreType.DMA,
        ],
    )
    cntk = pl.kernel(
        _sc_cnt_body,
        out_type=jax.ShapeDtypeStruct((NC, N_PAD, HID), jnp.float32),
        mesh=mesh,
        scratch_types=[
            pltpu.VMEM((N_CHUNKS_CNT, CHUNK_CNT), jnp.int32),
            pltpu.VMEM((CHUNK_CNT, HID), jnp.float32),
            pltpu.VMEM_SHARED((N_PAD, HID), jnp.float32),
            pltpu.SemaphoreType.DMA,
        ],
    )
    return agg, cntk


ROW_BLK = 1024
N_BLKS = N_PAD // ROW_BLK


def _combine_body(sums_ref, cnt_ref, x_ref, wlT_ref, b_ref, wrT_ref, o_ref):
    sm = sums_ref[0] + sums_ref[1]
    c = cnt_ref[0, :, :1] + cnt_ref[1, :, :1]
    mean = sm / jnp.maximum(c, 1.0)
    h = (jnp.dot(mean, wlT_ref[...], preferred_element_type=jnp.float32)
         + b_ref[...]
         + jnp.dot(x_ref[...], wrT_ref[...], preferred_element_type=jnp.float32))
    o_ref[...] = jnp.maximum(h, 0.0)


def _combine(sums, cnt, x, wlT, b_row, wrT):
    return pl.pallas_call(
        _combine_body,
        grid=(N_BLKS,),
        in_specs=[
            pl.BlockSpec((NC, ROW_BLK, HID), lambda i: (0, i, 0)),
            pl.BlockSpec((NC, ROW_BLK, HID), lambda i: (0, i, 0)),
            pl.BlockSpec((ROW_BLK, IN_CH), lambda i: (i, 0)),
            pl.BlockSpec((IN_CH, HID), lambda i: (0, 0)),
            pl.BlockSpec((1, HID), lambda i: (0, 0)),
            pl.BlockSpec((IN_CH, HID), lambda i: (0, 0)),
        ],
        out_specs=pl.BlockSpec((ROW_BLK, HID), lambda i: (i, 0)),
        out_shape=jax.ShapeDtypeStruct((N_PAD, HID), jnp.float32),
    )(sums, cnt, x, wlT, b_row, wrT)


def _spline_mix(h, swT_ref, ssT_ref, out_width):
    u = h * 2.5 + 5.5
    mf = jnp.floor(u)
    t = u - mf
    t2 = t * t
    t3 = t2 * t
    w = 1.0 - t
    seg0 = t3 * (1.0 / 6.0)
    seg1 = (-3.0 * t3 + 3.0 * t2 + 3.0 * t + 1.0) * (1.0 / 6.0)
    seg2 = (3.0 * t3 - 6.0 * t2 + 4.0) * (1.0 / 6.0)
    seg3 = (w * w * w) * (1.0 / 6.0)
    ss = ssT_ref[...]
    acc = jnp.zeros((h.shape[0], out_width), jnp.float32)
    for j in range(N_COEFF):
        d = mf - float(j)
        bj = jnp.where(
            d == 0.0, seg0,
            jnp.where(d == 1.0, seg1,
                      jnp.where(d == 2.0, seg2,
                                jnp.where(d == 3.0, seg3, 0.0))))
        acc = acc + jnp.dot(bj, swT_ref[j] * ss,
                            preferred_element_type=jnp.float32)
    return acc


def _silu(v):
    return v / (1.0 + jnp.exp(-v))


def _kan_body(h_ref, bw1T_ref, sw1T_ref, ss1T_ref, bw2T_ref, sw2T_ref,
              ss2T_ref, o_ref):
    h = h_ref[...]
    mid = jnp.dot(_silu(h), bw1T_ref[...], preferred_element_type=jnp.float32)
    mid = mid + _spline_mix(h, sw1T_ref, ss1T_ref, KAN_MID)
    out = jnp.dot(_silu(mid), bw2T_ref[...], preferred_element_type=jnp.float32)
    out = out + _spline_mix(mid, sw2T_ref, ss2T_ref, OUT_CH)
    o_ref[...] = out


def _kan(h, bw1T, sw1T, ss1T, bw2T, sw2T, ss2T):
    return pl.pallas_call(
        _kan_body,
        grid=(N_BLKS,),
        in_specs=[
            pl.BlockSpec((ROW_BLK, HID), lambda i: (i, 0)),
            pl.BlockSpec((HID, KAN_MID), lambda i: (0, 0)),
            pl.BlockSpec((N_COEFF, HID, KAN_MID), lambda i: (0, 0, 0)),
            pl.BlockSpec((HID, KAN_MID), lambda i: (0, 0)),
            pl.BlockSpec((KAN_MID, OUT_CH), lambda i: (0, 0)),
            pl.BlockSpec((N_COEFF, KAN_MID, OUT_CH), lambda i: (0, 0, 0)),
            pl.BlockSpec((KAN_MID, OUT_CH), lambda i: (0, 0)),
        ],
        out_specs=pl.BlockSpec((ROW_BLK, OUT_CH), lambda i: (i, 0)),
        out_shape=jax.ShapeDtypeStruct((N_PAD, OUT_CH), jnp.float32),
    )(h, bw1T, sw1T, ss1T, bw2T, sw2T, ss2T)


def kernel(x, edge_index, W1l, b1, W1r, W2l, b2, W2r, bw1, sw1, ss1,
           bw2, sw2, ss2):
    ei = edge_index.astype(jnp.int32)
    xp = jnp.concatenate([x, jnp.zeros((N_PAD - N_NODES, IN_CH), jnp.float32)])
    zrows = jnp.zeros((ROWS_PER_TILE, HID), jnp.float32)
    ones128 = jnp.ones((CHUNK_CNT, HID), jnp.float32)

    sc_agg, sc_cnt = _sc_kernels()
    pad = E_PAD - N_EDGES
    dst3 = jnp.concatenate([ei[1], jnp.full((pad,), N_NODES, jnp.int32)]
                           ).reshape(NC * NS, N_CHUNKS_CNT, CHUNK_CNT)
    slack = jnp.full((CHUNK,), N_NODES, jnp.int32)
    srcf = jnp.concatenate([ei[0], slack])
    dstf = jnp.concatenate([ei[1], slack])
    cnt = sc_cnt(dst3, zrows, ones128)
    sums1 = sc_agg(xp, srcf, dstf, zrows)
    h1 = _combine(sums1, cnt, xp, W1l.T, b1.reshape(1, HID), W1r.T)
    sums2 = sc_agg(h1, srcf, dstf, zrows)
    h2 = _combine(sums2, cnt, h1, W2l.T, b2.reshape(1, HID), W2r.T)
    out = _kan(h2, bw1.T, jnp.transpose(sw1, (2, 1, 0)), ss1.T,
               bw2.T, jnp.transpose(sw2, (2, 1, 0)), ss2.T)
    return out[:N_NODES]
